# baseline (speedup 1.0000x reference)
import functools

import jax
import jax.numpy as jnp
from jax.experimental import pallas as pl
from jax.experimental.pallas import tpu as pltpu


def _pipe_kernel(nc, chunk, K, table_ref, o_ref, bufs, in_sems, out_sems):
    B = o_ref.shape[0]

    def in_copy(c):
        k = c % K
        return pltpu.make_async_copy(
            table_ref.at[pl.ds(c * chunk, chunk), :], bufs.at[k], in_sems.at[k]
        )

    def out_copy(c, b):
        k = c % K
        return pltpu.make_async_copy(
            bufs.at[k], o_ref.at[b, pl.ds(c * chunk, chunk), :], out_sems.at[k]
        )

    in_copy(0).start()
    for c in range(nc):
        if c + 1 < nc:
            if c + 1 >= K:
                for b in range(B):
                    out_copy(c + 1 - K, b).wait()
            in_copy(c + 1).start()
        in_copy(c).wait()
        for b in range(B):
            out_copy(c, b).start()
    for c in range(max(0, nc - K), nc):
        for b in range(B):
            out_copy(c, b).wait()


def kernel(x, pos_embeddings):
    B, S = x.shape
    D = pos_embeddings.shape[1]
    nc = 16
    chunk = S // nc
    K = 4
    return pl.pallas_call(
        functools.partial(_pipe_kernel, nc, chunk, K),
        in_specs=[pl.BlockSpec(memory_space=pl.ANY)],
        out_specs=pl.BlockSpec(memory_space=pl.ANY),
        out_shape=jax.ShapeDtypeStruct((B, S, D), pos_embeddings.dtype),
        scratch_shapes=[
            pltpu.VMEM((K, S // nc, D), pos_embeddings.dtype),
            pltpu.SemaphoreType.DMA((K,)),
            pltpu.SemaphoreType.DMA((K,)),
        ],
    )(pos_embeddings)


# speedup vs baseline: 1.0544x; 1.0544x over previous
import functools

import jax
import jax.numpy as jnp
from jax.experimental import pallas as pl
from jax.experimental.pallas import tpu as pltpu


def _pipe_kernel(nc, chunk, K, table_ref, o_ref, bufs, in_sems, out_sems):
    B = o_ref.shape[0]

    def in_copy(c):
        k = c % K
        return pltpu.make_async_copy(
            table_ref.at[pl.ds(c * chunk, chunk), :], bufs.at[k], in_sems.at[k]
        )

    def out_copy(c, b):
        k = c % K
        return pltpu.make_async_copy(
            bufs.at[k], o_ref.at[b, pl.ds(c * chunk, chunk), :], out_sems.at[k]
        )

    in_copy(0).start()
    for c in range(nc):
        if c + 1 < nc:
            if c + 1 >= K:
                for b in range(B):
                    out_copy(c + 1 - K, b).wait()
            in_copy(c + 1).start()
        in_copy(c).wait()
        for b in range(B):
            out_copy(c, b).start()
    for c in range(max(0, nc - K), nc):
        for b in range(B):
            out_copy(c, b).wait()


def kernel(x, pos_embeddings):
    B, S = x.shape
    D = pos_embeddings.shape[1]
    nc = 4
    chunk = S // nc
    K = 3
    return pl.pallas_call(
        functools.partial(_pipe_kernel, nc, chunk, K),
        in_specs=[pl.BlockSpec(memory_space=pl.ANY)],
        out_specs=pl.BlockSpec(memory_space=pl.ANY),
        out_shape=jax.ShapeDtypeStruct((B, S, D), pos_embeddings.dtype),
        scratch_shapes=[
            pltpu.VMEM((K, S // nc, D), pos_embeddings.dtype),
            pltpu.SemaphoreType.DMA((K,)),
            pltpu.SemaphoreType.DMA((K,)),
        ],
    )(pos_embeddings)


# speedup vs baseline: 1.0608x; 1.0060x over previous
import functools

import jax
import jax.numpy as jnp
from jax.experimental import pallas as pl
from jax.experimental.pallas import tpu as pltpu


def _pipe_kernel(nc, chunk, K, table_ref, o_ref, bufs, in_sems, out_sems):
    B = o_ref.shape[0]

    def in_copy(c):
        k = c % K
        return pltpu.make_async_copy(
            table_ref.at[pl.ds(c * chunk, chunk), :], bufs.at[k], in_sems.at[k]
        )

    def out_copy(c, b):
        k = c % K
        return pltpu.make_async_copy(
            bufs.at[k], o_ref.at[b, pl.ds(c * chunk, chunk), :], out_sems.at[k]
        )

    in_copy(0).start()
    for c in range(nc):
        if c + 1 < nc:
            if c + 1 >= K:
                for b in range(B):
                    out_copy(c + 1 - K, b).wait()
            in_copy(c + 1).start()
        in_copy(c).wait()
        for b in range(B):
            out_copy(c, b).start()
    for c in range(max(0, nc - K), nc):
        for b in range(B):
            out_copy(c, b).wait()


def kernel(x, pos_embeddings):
    B, S = x.shape
    D = pos_embeddings.shape[1]
    nc = 2
    chunk = S // nc
    K = 2
    return pl.pallas_call(
        functools.partial(_pipe_kernel, nc, chunk, K),
        in_specs=[pl.BlockSpec(memory_space=pl.ANY)],
        out_specs=pl.BlockSpec(memory_space=pl.ANY),
        out_shape=jax.ShapeDtypeStruct((B, S, D), pos_embeddings.dtype),
        scratch_shapes=[
            pltpu.VMEM((K, S // nc, D), pos_embeddings.dtype),
            pltpu.SemaphoreType.DMA((K,)),
            pltpu.SemaphoreType.DMA((K,)),
        ],
    )(pos_embeddings)
